# baseline (speedup 1.0000x reference)
import functools

import jax
import jax.numpy as jnp
from jax import lax
from jax.experimental import pallas as pl
from jax.experimental.pallas import tpu as pltpu
from jax.experimental.pallas import tpu_sc as plsc

_EMBED = 64
_NW = 32
_GRP = 2
_NCK = 4


@functools.cache
def _build(batch, hist):
    b_per_w = batch // _NW
    ngroups = b_per_w // _GRP
    chunk = _GRP * hist // _NCK
    assert chunk <= 128 and ngroups % 2 == 0
    nidx = b_per_w * hist // chunk
    mesh = plsc.VectorSubcoreMesh(core_axis_name="c", subcore_axis_name="s")

    @functools.partial(
        pl.kernel,
        mesh=mesh,
        compiler_params=pltpu.CompilerParams(use_tc_tiling_on_sc=False),
        out_type=jax.ShapeDtypeStruct((batch, hist, _EMBED), jnp.float32),
        scratch_types=[
            pltpu.VMEM((nidx, chunk), jnp.int32),
            pltpu.VMEM((_GRP, hist, _EMBED), jnp.float32),
            pltpu.VMEM((_GRP, hist, _EMBED), jnp.float32),
            pltpu.SemaphoreType.DMA,
            pltpu.SemaphoreType.DMA,
            pltpu.SemaphoreType.DMA,
            pltpu.SemaphoreType.DMA,
        ],
    )
    def gather_kernel(idx_hbm, table_hbm, out_hbm, idx_v, buf_a, buf_b,
                      sem_ga, sem_gb, sem_wa, sem_wb):
        wid = lax.axis_index("s") * 2 + lax.axis_index("c")
        bbase = wid * b_per_w
        pltpu.sync_copy(idx_hbm.at[wid], idx_v)

        def chunk_copies(g, buf, sem):
            for c in range(_NCK):
                flat = c * chunk
                dst = buf.at[flat // hist, pl.ds(flat % hist, chunk)]
                yield table_hbm.at[idx_v.at[g * _NCK + c]], dst, sem

        def start_gathers(g, buf, sem):
            for src, dst, s in chunk_copies(g, buf, sem):
                pltpu.async_copy(src, dst, s)

        def wait_gathers(g, buf, sem):
            for src, dst, s in chunk_copies(g, buf, sem):
                pltpu.make_async_copy(src, dst, s).wait()

        def out_slice(g):
            return out_hbm.at[pl.ds(bbase + g * _GRP, _GRP)]

        start_gathers(0, buf_a, sem_ga)

        def body(i, carry):
            ga = 2 * i
            gb = 2 * i + 1

            wait_gathers(ga, buf_a, sem_ga)

            @pl.when(i > 0)
            def _():
                pltpu.make_async_copy(buf_b, out_slice(gb - 2), sem_wb).wait()

            start_gathers(gb, buf_b, sem_gb)
            pltpu.async_copy(buf_a, out_slice(ga), sem_wa)

            wait_gathers(gb, buf_b, sem_gb)
            pltpu.make_async_copy(buf_a, out_slice(ga), sem_wa).wait()

            @pl.when(ga + 2 < ngroups)
            def _():
                start_gathers(ga + 2, buf_a, sem_ga)

            pltpu.async_copy(buf_b, out_slice(gb), sem_wb)
            return carry

        lax.fori_loop(0, ngroups // 2, body, 0)
        pltpu.make_async_copy(buf_b, out_slice(ngroups - 1), sem_wb).wait()

    return gather_kernel


def kernel(x, position_embedding_table):
    b, h = x.shape
    chunk = _GRP * h // _NCK
    idx = x.astype(jnp.int32).reshape(_NW, b * h // (_NW * chunk), chunk)
    fn = _build(b, h)
    return fn(idx, position_embedding_table)
